# baseline (speedup 1.0000x reference)
import functools

import jax
import jax.numpy as jnp
from jax import lax
from jax.experimental import pallas as pl
from jax.experimental.pallas import tpu as pltpu
from jax.experimental.pallas import tpu_sc as plsc

N, K, M = 10000, 64, 200
KK = K * K
E_TOT = N * M

BN = 40


def _adj_body(sa_ref, sba_ref, adj_ref):
    x = sba_ref[...]
    m = jnp.max(x, axis=2)
    s = jnp.sum(jnp.exp(x - m[:, :, None]), axis=2)
    row_lse = jnp.log(s) + m
    sv = sa_ref[...]
    sm = jnp.max(sv, axis=1)
    ss = jnp.sum(jnp.exp(sv - sm[:, None]), axis=1)
    sa_lse = jnp.log(ss) + sm
    adj_ref[...] = sv - sa_lse[:, None] - row_lse


def _compute_adj(sa, sba):
    return pl.pallas_call(
        _adj_body,
        grid=(N // BN,),
        in_specs=[
            pl.BlockSpec((BN, K), lambda i: (i, 0)),
            pl.BlockSpec((BN, K, K), lambda i: (i, 0, 0)),
        ],
        out_specs=pl.BlockSpec((BN, K), lambda i: (i, 0)),
        out_shape=jax.ShapeDtypeStruct((N, K), jnp.float32),
    )(sa, sba)


CHUNK = 4000
N_PER_CHUNK = CHUNK // M
NUM_CHUNKS = E_TOT // CHUNK
NV = CHUNK // 16
NW = 32
CHUNKS_PER_W = -(-NUM_CHUNKS // NW)


def _gather_body(sba_hbm, adj_hbm, a_hbm, b_hbm, out_hbm,
                 a_v, b_v, adj_v, idx_v, vals_v, acc_v, sem):
    wid = lax.axis_index("s") * 2 + lax.axis_index("c")

    def do_chunk(k, _):
        cid = wid + NW * k

        @pl.when(cid < NUM_CHUNKS)
        def _():
            base = cid * CHUNK
            pltpu.sync_copy(a_hbm.at[pl.ds(base, CHUNK)], a_v)
            pltpu.sync_copy(b_hbm.at[pl.ds(base, CHUNK)], b_v)
            pltpu.sync_copy(
                adj_hbm.at[pl.ds(cid * (N_PER_CHUNK * K), N_PER_CHUNK * K)],
                adj_v)
            n_base = cid * N_PER_CHUNK

            def idx_body(j, _):
                e16 = j * 16 + lax.iota(jnp.int32, 16)
                nloc = lax.shift_right_logical(e16 * 10486, 21)
                a16 = a_v[pl.ds(j * 16, 16)]
                b16 = b_v[pl.ds(j * 16, 16)]
                idx_v[pl.ds(j * 16, 16)] = (n_base + nloc) * KK + a16 * K + b16
                acc_v[pl.ds(j * 16, 16)] = plsc.load_gather(
                    adj_v, [nloc * K + a16])
                return 0

            lax.fori_loop(0, NV, idx_body, 0)
            pltpu.async_copy(sba_hbm.at[idx_v], vals_v, sem).wait()

            def add_body(j, _):
                sl = pl.ds(j * 16, 16)
                acc_v[sl] = acc_v[sl] + vals_v[sl]
                return 0

            lax.fori_loop(0, NV, add_body, 0)
            pltpu.sync_copy(acc_v, out_hbm.at[pl.ds(base, CHUNK)])

        return 0

    lax.fori_loop(0, CHUNKS_PER_W, do_chunk, 0)


def _gather_combine(sba_flat, adj_flat, a_flat, b_flat):
    mesh = plsc.VectorSubcoreMesh(core_axis_name="c", subcore_axis_name="s")
    return pl.kernel(
        _gather_body,
        out_type=jax.ShapeDtypeStruct((E_TOT,), jnp.float32),
        mesh=mesh,
        compiler_params=pltpu.CompilerParams(needs_layout_passes=False),
        scratch_types=[
            pltpu.VMEM((CHUNK,), jnp.int32),
            pltpu.VMEM((CHUNK,), jnp.int32),
            pltpu.VMEM((N_PER_CHUNK * K,), jnp.float32),
            pltpu.VMEM((CHUNK,), jnp.int32),
            pltpu.VMEM((CHUNK,), jnp.float32),
            pltpu.VMEM((CHUNK,), jnp.float32),
            pltpu.SemaphoreType.DMA,
        ],
    )(sba_flat, adj_flat, a_flat, b_flat)


def kernel(a, b, sa, sba):
    a = a.reshape(-1).astype(jnp.int32)
    b = b.reshape(-1).astype(jnp.int32)
    adj = _compute_adj(sa, sba)
    return _gather_combine(sba.reshape(-1), adj.reshape(-1), a, b)


# speedup vs baseline: 2.8406x; 2.8406x over previous
import jax
import jax.numpy as jnp
from jax import lax
from jax.experimental import pallas as pl
from jax.experimental.pallas import tpu as pltpu
from jax.experimental.pallas import tpu_sc as plsc

N, K, M = 10000, 64, 200
KK = K * K
E_TOT = N * M

BNL = 512
NB = N // BNL + 1
NPAD = NB * BNL


def _adj_body(sa_ref, sba_ref, adj_ref, flat_ref):
    x = sba_ref[...]
    x3 = x.reshape(K, K, BNL)
    m = jnp.max(x3, axis=1)
    s = jnp.sum(jnp.exp(x3 - m[:, None, :]), axis=1)
    row_lse = jnp.log(s) + m
    sv = sa_ref[...]
    sm = jnp.max(sv, axis=0)
    ss = jnp.sum(jnp.exp(sv - sm[None, :]), axis=0)
    sa_lse = jnp.log(ss) + sm
    adjv = sv - sa_lse[None, :] - row_lse
    adjp = jnp.concatenate(
        [adjv.T, jnp.zeros((BNL, 128 - K), jnp.float32)], axis=1)
    adj_ref[...] = adjp.reshape(-1)
    flat_ref[...] = x.T.reshape(-1)


def _compute_adj_flat(sa_t, sba_t):
    return pl.pallas_call(
        _adj_body,
        grid=(NB,),
        in_specs=[
            pl.BlockSpec((K, BNL), lambda i: (0, i)),
            pl.BlockSpec((KK, BNL), lambda i: (0, i)),
        ],
        out_specs=[
            pl.BlockSpec((BNL * 128,), lambda i: (i,)),
            pl.BlockSpec((BNL * KK,), lambda i: (i,)),
        ],
        out_shape=[
            jax.ShapeDtypeStruct((NPAD * 128,), jnp.float32),
            jax.ShapeDtypeStruct((NPAD * KK,), jnp.float32),
        ],
    )(sa_t, sba_t)


CHUNK = 4000
N_PER_CHUNK = CHUNK // M
NUM_CHUNKS = E_TOT // CHUNK
NV = CHUNK // 16
NW = 32
CHUNKS_PER_W = -(-NUM_CHUNKS // NW)


def _gather_body(sba_hbm, adj_hbm, a_hbm, b_hbm, out_hbm,
                 a_v, b_v, adj_v, idx_v, vals_v, acc_v, sem):
    wid = lax.axis_index("s") * 2 + lax.axis_index("c")

    def do_chunk(k, _):
        cid = wid + NW * k

        @pl.when(cid < NUM_CHUNKS)
        def _():
            base = cid * CHUNK
            pltpu.sync_copy(a_hbm.at[pl.ds(base, CHUNK)], a_v)
            pltpu.sync_copy(b_hbm.at[pl.ds(base, CHUNK)], b_v)
            pltpu.sync_copy(
                adj_hbm.at[pl.ds(cid * (N_PER_CHUNK * 128),
                                 N_PER_CHUNK * 128)],
                adj_v)
            n_base = cid * N_PER_CHUNK

            def idx_body(j, _):
                e16 = j * 16 + lax.iota(jnp.int32, 16)
                nloc = lax.shift_right_logical(e16 * 10486, 21)
                a16 = a_v[pl.ds(j * 16, 16)]
                b16 = b_v[pl.ds(j * 16, 16)]
                idx_v[pl.ds(j * 16, 16)] = (n_base + nloc) * KK + a16 * K + b16
                acc_v[pl.ds(j * 16, 16)] = plsc.load_gather(
                    adj_v, [nloc * 128 + a16])
                return 0

            lax.fori_loop(0, NV, idx_body, 0)
            pltpu.async_copy(sba_hbm.at[idx_v], vals_v, sem).wait()

            def add_body(j, _):
                sl = pl.ds(j * 16, 16)
                acc_v[sl] = acc_v[sl] + vals_v[sl]
                return 0

            lax.fori_loop(0, NV, add_body, 0)
            pltpu.sync_copy(acc_v, out_hbm.at[pl.ds(base, CHUNK)])

        return 0

    lax.fori_loop(0, CHUNKS_PER_W, do_chunk, 0)


def _gather_combine(sba_flat, adj_flat, a_flat, b_flat):
    mesh = plsc.VectorSubcoreMesh(core_axis_name="c", subcore_axis_name="s")
    return pl.kernel(
        _gather_body,
        out_type=jax.ShapeDtypeStruct((E_TOT,), jnp.float32),
        mesh=mesh,
        scratch_types=[
            pltpu.VMEM((CHUNK,), jnp.int32),
            pltpu.VMEM((CHUNK,), jnp.int32),
            pltpu.VMEM((N_PER_CHUNK * 128,), jnp.float32),
            pltpu.VMEM((CHUNK,), jnp.int32),
            pltpu.VMEM((CHUNK,), jnp.float32),
            pltpu.VMEM((CHUNK,), jnp.float32),
            pltpu.SemaphoreType.DMA,
        ],
        compiler_params=pltpu.CompilerParams(needs_layout_passes=False),
    )(sba_flat, adj_flat, a_flat, b_flat)


def kernel(a, b, sa, sba):
    a_flat = a.reshape(-1).astype(jnp.int32)
    b_flat = b.reshape(-1).astype(jnp.int32)
    sa_t = sa.T
    sba_t = jnp.transpose(sba, (1, 2, 0)).reshape(KK, N)
    adj_flat, sba_flat = _compute_adj_flat(sa_t, sba_t)
    return _gather_combine(sba_flat, adj_flat, a_flat, b_flat)


# speedup vs baseline: 3.6620x; 1.2891x over previous
import jax
import jax.numpy as jnp
from jax import lax
from jax.experimental import pallas as pl
from jax.experimental.pallas import tpu as pltpu
from jax.experimental.pallas import tpu_sc as plsc

N, K, M = 10000, 64, 200
KK = K * K
E_TOT = N * M

BNL = 512
NB = N // BNL + 1
NPAD = NB * BNL


def _adj_body(sa_ref, sba_ref, adj_ref, flat_ref):
    x = sba_ref[...]
    x3 = x.reshape(K, K, BNL)
    m = jnp.max(x3, axis=1)
    s = jnp.sum(jnp.exp(x3 - m[:, None, :]), axis=1)
    row_lse = jnp.log(s) + m
    sv = sa_ref[...]
    sm = jnp.max(sv, axis=0)
    ss = jnp.sum(jnp.exp(sv - sm[None, :]), axis=0)
    sa_lse = jnp.log(ss) + sm
    adjv = sv - sa_lse[None, :] - row_lse
    adjp = jnp.concatenate(
        [adjv.T, jnp.zeros((BNL, 128 - K), jnp.float32)], axis=1)
    adj_ref[...] = adjp.reshape(-1)
    flat_ref[...] = x.T.reshape(-1)


def _compute_adj_flat(sa_t, sba_t):
    return pl.pallas_call(
        _adj_body,
        grid=(NB,),
        in_specs=[
            pl.BlockSpec((K, BNL), lambda i: (0, i)),
            pl.BlockSpec((KK, BNL), lambda i: (0, i)),
        ],
        out_specs=[
            pl.BlockSpec((BNL * 128,), lambda i: (i,)),
            pl.BlockSpec((BNL * KK,), lambda i: (i,)),
        ],
        out_shape=[
            jax.ShapeDtypeStruct((NPAD * 128,), jnp.float32),
            jax.ShapeDtypeStruct((NPAD * KK,), jnp.float32),
        ],
    )(sa_t, sba_t)


CHUNK = 4000
N_PER_CHUNK = CHUNK // M
NUM_CHUNKS = E_TOT // CHUNK
NV = CHUNK // 16
NW = 32
CHUNKS_PER_W = -(-NUM_CHUNKS // NW)


AW = N_PER_CHUNK * 128


def _gather_body(sba_hbm, adj_hbm, a_hbm, b_hbm, out_hbm,
                 a_v, b_v, adj_v, idx_v, vals_v, acc_v, ldsem, gsem):
    wid = lax.axis_index("s") * 2 + lax.axis_index("c")

    def phase_a(k, p):
        cid = wid + NW * k

        @pl.when(cid < NUM_CHUNKS)
        def _():
            base = cid * CHUNK
            o = p * CHUNK
            c1 = pltpu.make_async_copy(
                a_hbm.at[pl.ds(base, CHUNK)], a_v.at[pl.ds(o, CHUNK)], ldsem)
            c2 = pltpu.make_async_copy(
                b_hbm.at[pl.ds(base, CHUNK)], b_v.at[pl.ds(o, CHUNK)], ldsem)
            c3 = pltpu.make_async_copy(
                adj_hbm.at[pl.ds(cid * AW, AW)],
                adj_v.at[pl.ds(p * AW, AW)], ldsem)
            c1.start(); c2.start(); c3.start()
            c1.wait(); c2.wait(); c3.wait()
            n_base = cid * N_PER_CHUNK

            @plsc.parallel_loop(0, NV, unroll=8)
            def _(j):
                e16 = j * 16 + lax.iota(jnp.int32, 16)
                nloc = lax.shift_right_logical(e16 * 10486, 21)
                a16 = a_v[pl.ds(o + j * 16, 16)]
                b16 = b_v[pl.ds(o + j * 16, 16)]
                idx_v[pl.ds(o + j * 16, 16)] = (
                    (n_base + nloc) * KK + a16 * K + b16)
                acc_v[pl.ds(o + j * 16, 16)] = plsc.load_gather(
                    adj_v, [p * AW + nloc * 128 + a16])

            pltpu.make_async_copy(
                sba_hbm.at[idx_v.at[pl.ds(o, CHUNK)]],
                vals_v.at[pl.ds(o, CHUNK)], gsem).start()

    def phase_b(k, p):
        cid = wid + NW * k

        @pl.when(cid < NUM_CHUNKS)
        def _():
            o = p * CHUNK
            pltpu.make_async_copy(
                sba_hbm.at[idx_v.at[pl.ds(o, CHUNK)]],
                vals_v.at[pl.ds(o, CHUNK)], gsem).wait()

            @plsc.parallel_loop(0, NV, unroll=8)
            def _(j):
                sl = pl.ds(o + j * 16, 16)
                acc_v[sl] = acc_v[sl] + vals_v[sl]

            pltpu.sync_copy(acc_v.at[pl.ds(o, CHUNK)],
                            out_hbm.at[pl.ds(cid * CHUNK, CHUNK)])

    phase_a(0, 0)
    for k in range(1, CHUNKS_PER_W):
        phase_a(k, k % 2)
        phase_b(k - 1, (k - 1) % 2)
    phase_b(CHUNKS_PER_W - 1, (CHUNKS_PER_W - 1) % 2)


def _gather_combine(sba_flat, adj_flat, a_flat, b_flat):
    mesh = plsc.VectorSubcoreMesh(core_axis_name="c", subcore_axis_name="s")
    return pl.kernel(
        _gather_body,
        out_type=jax.ShapeDtypeStruct((E_TOT,), jnp.float32),
        mesh=mesh,
        scratch_types=[
            pltpu.VMEM((2 * CHUNK,), jnp.int32),
            pltpu.VMEM((2 * CHUNK,), jnp.int32),
            pltpu.VMEM((2 * AW,), jnp.float32),
            pltpu.VMEM((2 * CHUNK,), jnp.int32),
            pltpu.VMEM((2 * CHUNK,), jnp.float32),
            pltpu.VMEM((2 * CHUNK,), jnp.float32),
            pltpu.SemaphoreType.DMA,
            pltpu.SemaphoreType.DMA,
        ],
        compiler_params=pltpu.CompilerParams(needs_layout_passes=False),
    )(sba_flat, adj_flat, a_flat, b_flat)


def kernel(a, b, sa, sba):
    a_flat = a.reshape(-1).astype(jnp.int32)
    b_flat = b.reshape(-1).astype(jnp.int32)
    sa_t = sa.T
    sba_t = jnp.transpose(sba, (1, 2, 0)).reshape(KK, N)
    adj_flat, sba_flat = _compute_adj_flat(sa_t, sba_t)
    return _gather_combine(sba_flat, adj_flat, a_flat, b_flat)
